# Initial kernel scaffold; baseline (speedup 1.0000x reference)
import jax
import jax.numpy as jnp
from jax.experimental import pallas as pl


def kernel(i, idx_i, p):
    raise NotImplementedError("write your pallas kernel here")


# speedup vs baseline: 2.2896x; 2.2896x over previous
import functools

import jax
import jax.numpy as jnp
from jax import lax
from jax.experimental import pallas as pl
from jax.experimental.pallas import tpu as pltpu
from jax.experimental.pallas import tpu_sc as plsc

N_I = 819200
N_P = 100000
D = 64
L = 16
NC = 2
NS = 16
GROUPS = D // L
PASSES = GROUPS // NC
ROWS_PER_TILE = N_I // NS
CH = 1024
NCH = ROWS_PER_TILE // CH
SCB = 128
P_PER_TILE = N_P // NS
PCH = 625
NPCH = P_PER_TILE // PCH

_mesh = plsc.VectorSubcoreMesh(core_axis_name="c", subcore_axis_name="s")


@functools.partial(
    pl.kernel,
    mesh=_mesh,
    out_type=jax.ShapeDtypeStruct((N_P, D), jnp.float32),
    scratch_types=[
        pltpu.VMEM((CH, L), jnp.float32),
        pltpu.VMEM((CH // SCB, SCB), jnp.int32),
        pltpu.VMEM((PCH, L), jnp.float32),
        pltpu.VMEM_SHARED((N_P, L), jnp.float32),
    ],
    compiler_params=pltpu.CompilerParams(use_tc_tiling_on_sc=False),
)
def _scatter_add(i_hbm, idx_hbm, p_hbm, out_hbm, ibuf, idxbuf, pbuf, acc):
    cid = lax.axis_index("c")
    sid = lax.axis_index("s")
    row0 = sid * ROWS_PER_TILE
    blk0 = sid * (ROWS_PER_TILE // SCB)
    prow0 = sid * P_PER_TILE

    for pz in range(PASSES):
        c0 = (pz * NC + cid) * L

        for q in range(NPCH):
            r = prow0 + q * PCH
            pltpu.sync_copy(p_hbm.at[pl.ds(r, PCH), pl.ds(c0, L)], pbuf)
            pltpu.sync_copy(pbuf, acc.at[pl.ds(r, PCH)])
        plsc.subcore_barrier()

        def chunk(ch, carry):
            r = row0 + ch * CH
            b = blk0 + ch * (CH // SCB)
            pltpu.sync_copy(idx_hbm.at[pl.ds(b, CH // SCB)], idxbuf)
            pltpu.sync_copy(i_hbm.at[pl.ds(r, CH), pl.ds(c0, L)], ibuf)
            for j in range(CH // SCB):
                pltpu.sync_copy(ibuf.at[pl.ds(j * SCB, SCB)],
                                acc.at[idxbuf.at[j]], add=True)
            return carry

        lax.fori_loop(0, NCH, chunk, 0)
        plsc.subcore_barrier()

        for q in range(NPCH):
            r = prow0 + q * PCH
            pltpu.sync_copy(acc.at[pl.ds(r, PCH)], pbuf)
            pltpu.sync_copy(pbuf, out_hbm.at[pl.ds(r, PCH), pl.ds(c0, L)])


@jax.jit
def kernel(i, idx_i, p):
    idx = jnp.asarray(idx_i, jnp.int32).reshape(N_I // SCB, SCB)
    return _scatter_add(i, idx, p)


# speedup vs baseline: 2.5509x; 1.1141x over previous
import functools

import jax
import jax.numpy as jnp
from jax import lax
from jax.experimental import pallas as pl
from jax.experimental.pallas import tpu as pltpu
from jax.experimental.pallas import tpu_sc as plsc

N_I = 819200
N_P = 100000
D = 64
L = 16
NC = 2
NS = 16
GROUPS = D // L
PASSES = GROUPS // NC
ROWS_PER_TILE = N_I // NS
CH = 512
NCH = ROWS_PER_TILE // CH
SCB = 128
K = CH // SCB
P_PER_TILE = N_P // NS
PCH = 625
NPCH = P_PER_TILE // PCH

_mesh = plsc.VectorSubcoreMesh(core_axis_name="c", subcore_axis_name="s")


@functools.partial(
    pl.kernel,
    mesh=_mesh,
    out_type=jax.ShapeDtypeStruct((N_P, D), jnp.float32),
    scratch_types=[
        pltpu.VMEM((CH, L), jnp.float32),
        pltpu.VMEM((CH, L), jnp.float32),
        pltpu.VMEM((K, SCB), jnp.int32),
        pltpu.VMEM((K, SCB), jnp.int32),
        pltpu.VMEM((PCH, L), jnp.float32),
        pltpu.VMEM_SHARED((N_P, L), jnp.float32),
        pltpu.SemaphoreType.DMA,
        pltpu.SemaphoreType.DMA,
        pltpu.SemaphoreType.DMA,
        pltpu.SemaphoreType.DMA,
    ],
    compiler_params=pltpu.CompilerParams(use_tc_tiling_on_sc=False),
)
def _scatter_add(i_hbm, idx_hbm, p_hbm, out_hbm,
                 ib0, ib1, xb0, xb1, pbuf, acc,
                 ls0, ls1, ss0, ss1):
    cid = lax.axis_index("c")
    sid = lax.axis_index("s")
    row0 = sid * ROWS_PER_TILE
    blk0 = sid * (ROWS_PER_TILE // SCB)
    prow0 = sid * P_PER_TILE

    ibufs, xbufs = (ib0, ib1), (xb0, xb1)
    lsems, ssems = (ls0, ls1), (ss0, ss1)

    def load_start(b, c, c0):
        pltpu.async_copy(idx_hbm.at[pl.ds(blk0 + c * K, K)], xbufs[b],
                         lsems[b])
        pltpu.async_copy(i_hbm.at[pl.ds(row0 + c * CH, CH), pl.ds(c0, L)],
                         ibufs[b], lsems[b])

    def load_wait(b, c, c0):
        pltpu.make_async_copy(idx_hbm.at[pl.ds(blk0 + c * K, K)], xbufs[b],
                              lsems[b]).wait()
        pltpu.make_async_copy(i_hbm.at[pl.ds(row0 + c * CH, CH),
                                       pl.ds(c0, L)],
                              ibufs[b], lsems[b]).wait()

    def scatter_start(b):
        for j in range(K):
            pltpu.async_copy(ibufs[b].at[pl.ds(j * SCB, SCB)],
                             acc.at[xbufs[b].at[j]], ssems[b], add=True)

    def scatter_wait(b):
        for j in range(K):
            pltpu.make_async_copy(ibufs[b].at[pl.ds(j * SCB, SCB)],
                                  acc.at[xbufs[b].at[j]], ssems[b]).wait()

    for pz in range(PASSES):
        c0 = (pz * NC + cid) * L

        for q in range(NPCH):
            r = prow0 + q * PCH
            pltpu.sync_copy(p_hbm.at[pl.ds(r, PCH), pl.ds(c0, L)], pbuf)
            pltpu.sync_copy(pbuf, acc.at[pl.ds(r, PCH)])
        plsc.subcore_barrier()

        load_start(0, 0, c0)
        load_start(1, 1, c0)
        load_wait(0, 0, c0)
        scatter_start(0)

        def body(t, carry):
            ca = 2 * t
            load_wait(1, ca + 1, c0)
            scatter_start(1)
            scatter_wait(0)
            load_start(0, ca + 2, c0)
            load_wait(0, ca + 2, c0)
            scatter_start(0)
            scatter_wait(1)
            load_start(1, ca + 3, c0)
            return carry

        lax.fori_loop(0, (NCH - 2) // 2, body, 0)
        last = jnp.int32(NCH - 1)
        load_wait(1, last, c0)
        scatter_start(1)
        scatter_wait(0)
        scatter_wait(1)
        plsc.subcore_barrier()

        for q in range(NPCH):
            r = prow0 + q * PCH
            pltpu.sync_copy(acc.at[pl.ds(r, PCH)], pbuf)
            pltpu.sync_copy(pbuf, out_hbm.at[pl.ds(r, PCH), pl.ds(c0, L)])


@jax.jit
def kernel(i, idx_i, p):
    idx = jnp.asarray(idx_i, jnp.int32).reshape(N_I // SCB, SCB)
    return _scatter_add(i, idx, p)
